# baseline (speedup 1.0000x reference)
import jax
import jax.numpy as jnp
from jax import lax
from jax.experimental import pallas as pl
from jax.experimental.pallas import tpu as pltpu
from jax.experimental.pallas import tpu_sc as plsc

N = 10000
P = 320000
D = 128
K = 64
B = 5
RI = 3
RF = 2

NC = 2
NS = 16
NW = NC * NS
Q = 80
PAIRS_PER_WORKER = P // NW
N_BATCH = PAIRS_PER_WORKER // Q
GRP = 6
NGRP = 21
RING = 3
N_MACRO = 20
GT = 4000
STRIPE = 624
ZCH = 8
TAIL = N - NS * STRIPE

SC_G_SHAPE = (P // 2, D)
SC_G_DTYPE = jnp.int32
SC_XJT_SHAPE = (N, D)


def _ssp(x):
    return jnp.maximum(x, 0.0) + jnp.log1p(jnp.exp(-jnp.abs(x))) - 0.6931471805599453


def _pack_bf16(v):
    lo = jax.lax.bitcast_convert_type(v[:, :D // 2].astype(jnp.bfloat16), jnp.uint16).astype(jnp.int32)
    hi = jax.lax.bitcast_convert_type(v[:, D // 2:].astype(jnp.bfloat16), jnp.uint16).astype(jnp.int32)
    w = lo | (hi << 16)
    w3 = w.reshape(w.shape[0] // 2, 2, D // 2)
    return jnp.concatenate([w3[:, 0, :], w3[:, 1, :]], axis=-1)


def _g_body(cut_ref, rbf_ref, wr_ref, *g_refs):
    desc = cut_ref[...] * rbf_ref[...]
    for blk in range(B):
        g_refs[blk][...] = _pack_bf16(
            jnp.dot(desc, wr_ref[blk], preferred_element_type=jnp.float32))


def _g_call(cutoffs, rbfs, wr_all):
    T = GT
    grid = P // T
    return pl.pallas_call(
        _g_body,
        grid=(grid,),
        in_specs=[
            pl.BlockSpec((T, 1), lambda i: (i, 0)),
            pl.BlockSpec((T, K), lambda i: (i, 0)),
            pl.BlockSpec((B, K, D), lambda i: (0, 0, 0)),
        ],
        out_specs=[pl.BlockSpec((T // 2, D), lambda i: (i, 0))] * B,
        out_shape=[jax.ShapeDtypeStruct((P // 2, D), jnp.int32)] * B,
    )(cutoffs, rbfs, wr_all)


def _dense_a_body(x_ref, wi_ref, bi_ref, wj_ref, bj_ref, xi_ref, xjt_ref):
    xa = _ssp(x_ref[...])
    xi_ref[...] = jnp.dot(xa, wi_ref[...], preferred_element_type=jnp.float32) + bi_ref[...]
    xjt_ref[...] = jnp.dot(xa, wj_ref[...], preferred_element_type=jnp.float32) + bj_ref[...]


def _dense_a_call(x, wi, bi, wj, bj):
    return pl.pallas_call(
        _dense_a_body,
        out_shape=(
            jax.ShapeDtypeStruct((N, D), jnp.float32),
            jax.ShapeDtypeStruct((N, D), jnp.float32),
        ),
    )(x, wi, bi, wj, bj)


def _sc_body(g_hbm, xjt_hbm, idxc_hbm, out_hbm,
             idxj_s0, idxj_s1, idxj_s2,
             idxi_s0, idxi_s1, idxi_s2,
             idx_g, rows_v, g_v, zero_v, acc_sh,
             sem_idx, sem_r, sem_g, sem_s):
    idxj_s = (idxj_s0, idxj_s1, idxj_s2)
    idxi_s = (idxi_s0, idxi_s1, idxi_s2)
    c = lax.axis_index("c")
    s = lax.axis_index("s")
    wid = c * NS + s
    base = wid * PAIRS_PER_WORKER
    gbase = wid * (PAIRS_PER_WORKER // 2)

    def _zf(i, _):
        for j in range(D // 16):
            zero_v[i, pl.ds(j * 16, 16)] = jnp.zeros((16,), jnp.float32)
        return 0
    lax.fori_loop(0, ZCH, _zf, 0)
    for k in range(STRIPE // ZCH):
        pltpu.sync_copy(zero_v, acc_sh.at[pl.ds(s * STRIPE + k * ZCH, ZCH)])

    @pl.when(s == NS - 1)
    def _zero_tail():
        for k in range(TAIL // ZCH):
            pltpu.sync_copy(zero_v, acc_sh.at[pl.ds(NS * STRIPE + k * ZCH, ZCH)])

    plsc.subcore_barrier()

    def _start_idx(grp):
        pltpu.async_copy(idxc_hbm.at[wid, grp], idx_g.at[grp % 2], sem_idx)

    def _wait_idx():
        pltpu.make_async_copy(idxc_hbm.at[0, 0], idx_g.at[0], sem_idx).wait()

    def _stage(bn, ring):
        slot = (bn // GRP) % 2
        bb = bn % GRP
        for t in range(Q // 16):
            sl = pl.ds(t * 16, 16)
            idxj_s[ring][sl] = idx_g[slot, bb, 0, sl]
            idxi_s[ring][sl] = idx_g[slot, bb, 1, sl]

    def _start_loads(bn, ring, gslot):
        pltpu.async_copy(xjt_hbm.at[idxj_s[ring]], rows_v.at[ring], sem_r)
        pltpu.async_copy(g_hbm.at[pl.ds(gbase + bn * (Q // 2), Q // 2)],
                         g_v.at[gslot], sem_g)

    def _wait_loads(ring, gslot):
        pltpu.make_async_copy(xjt_hbm.at[idxj_s[0]], rows_v.at[ring], sem_r).wait()
        pltpu.make_async_copy(g_hbm.at[pl.ds(0, Q // 2)], g_v.at[gslot], sem_g).wait()

    def _start_scatter(ring):
        pltpu.async_copy(rows_v.at[ring], acc_sh.at[idxi_s[ring]], sem_s, add=True)

    def _wait_scatter(ring):
        pltpu.make_async_copy(rows_v.at[ring], acc_sh.at[idxi_s[0]], sem_s).wait()

    def _mul(ring, gslot):
        @plsc.parallel_loop(0, Q // 2, unroll=4)
        def _rowpair(m):
            for sub in range(2):
                r = m * 2 + sub
                for j in range(D // 32):
                    wg = g_v[gslot, m, pl.ds(64 * sub + 16 * j, 16)]
                    glo = plsc.bitcast(wg << 16, jnp.float32)
                    ghi = plsc.bitcast(wg & -65536, jnp.float32)
                    lo = pl.ds(16 * j, 16)
                    hi = pl.ds(64 + 16 * j, 16)
                    rows_v[ring, r, lo] = rows_v[ring, r, lo] * glo
                    rows_v[ring, r, hi] = rows_v[ring, r, hi] * ghi

    _start_idx(0)
    _wait_idx()
    _stage(0, 0)
    _start_loads(0, 0, 0)

    def _macro(m, _):
        for b in range(GRP):
            bi = m * GRP + b
            ring = b % RING
            gslot = b % 2
            ring_n = (b + 1) % RING
            gslot_n = (b + 1) % 2
            if b == 0:
                _start_idx(m + 1)
            if b < 2:
                @pl.when(m > 0)
                def _ws():
                    _wait_scatter(ring_n)
            else:
                if b == GRP - 1:
                    _wait_idx()
                _wait_scatter(ring_n)
            _stage(bi + 1, ring_n)
            _start_loads(bi + 1, ring_n, gslot_n)
            _wait_loads(ring, gslot)
            _mul(ring, gslot)
            _start_scatter(ring)
        return 0

    lax.fori_loop(0, N_MACRO, _macro, 0)

    for bi in range(N_MACRO * GRP, N_BATCH):
        ring = bi % RING
        gslot = bi % 2
        if bi + 1 < N_BATCH:
            _wait_scatter((bi + 1) % RING)
            _stage(bi + 1, (bi + 1) % RING)
            _start_loads(bi + 1, (bi + 1) % RING, (bi + 1) % 2)
        _wait_loads(ring, gslot)
        _mul(ring, gslot)
        _start_scatter(ring)
    for ring in ((N_BATCH - 3) % RING, (N_BATCH - 2) % RING, (N_BATCH - 1) % RING):
        _wait_scatter(ring)

    plsc.subcore_barrier()
    pltpu.sync_copy(acc_sh.at[pl.ds(s * STRIPE, STRIPE)],
                    out_hbm.at[c, pl.ds(s * STRIPE, STRIPE)])

    @pl.when(s == NS - 1)
    def _out_tail():
        pltpu.sync_copy(acc_sh.at[pl.ds(NS * STRIPE, TAIL)],
                        out_hbm.at[c, pl.ds(NS * STRIPE, TAIL)])


def _sc_call(g, xjt, idx_cat):
    mesh = plsc.VectorSubcoreMesh(core_axis_name="c", subcore_axis_name="s",
                                  num_cores=NC, num_subcores=NS)
    return pl.kernel(
        _sc_body,
        out_type=jax.ShapeDtypeStruct((NC, N, D), jnp.float32),
        mesh=mesh,
        compiler_params=pltpu.CompilerParams(needs_layout_passes=False),
        scratch_types=[
            pltpu.VMEM((Q,), jnp.int32),
            pltpu.VMEM((Q,), jnp.int32),
            pltpu.VMEM((Q,), jnp.int32),
            pltpu.VMEM((Q,), jnp.int32),
            pltpu.VMEM((Q,), jnp.int32),
            pltpu.VMEM((Q,), jnp.int32),
            pltpu.VMEM((2, GRP, 2, Q), jnp.int32),
            pltpu.VMEM((RING, Q, D), jnp.float32),
            pltpu.VMEM((2, Q // 2, D), jnp.int32),
            pltpu.VMEM((ZCH, D), jnp.float32),
            pltpu.VMEM_SHARED((N, D), jnp.float32),
            pltpu.SemaphoreType.DMA,
            pltpu.SemaphoreType.DMA,
            pltpu.SemaphoreType.DMA,
            pltpu.SemaphoreType.DMA,
        ],
    )(g, xjt, idx_cat)


def _finish_block(x, xi, acc_ref, wri_ref, bri_ref, wout_ref, bout_ref,
                  u_ref, wrf_ref, brf_ref):
    m = xi + acc_ref[0] + acc_ref[1]
    for r in range(RI):
        y = jnp.dot(_ssp(m), wri_ref[2 * r], preferred_element_type=jnp.float32) + bri_ref[2 * r]
        y = jnp.dot(_ssp(y), wri_ref[2 * r + 1], preferred_element_type=jnp.float32) + bri_ref[2 * r + 1]
        m = m + y
    m = _ssp(m)
    x = u_ref[...] * x + jnp.dot(m, wout_ref[...], preferred_element_type=jnp.float32) + bout_ref[...]
    for r in range(RF):
        y = jnp.dot(_ssp(x), wrf_ref[2 * r], preferred_element_type=jnp.float32) + brf_ref[2 * r]
        y = jnp.dot(_ssp(y), wrf_ref[2 * r + 1], preferred_element_type=jnp.float32) + brf_ref[2 * r + 1]
        x = x + y
    return x


def _dense_b_body(x_ref, xi_ref, acc_ref, wri_ref, bri_ref, wout_ref, bout_ref,
                  u_ref, wrf_ref, brf_ref, out_ref):
    out_ref[...] = _finish_block(x_ref[...], xi_ref[...], acc_ref, wri_ref,
                                 bri_ref, wout_ref, bout_ref, u_ref, wrf_ref,
                                 brf_ref)


def _dense_b_call(x, xi, acc, wri, bri, wout, bout, u, wrf, brf):
    return pl.pallas_call(
        _dense_b_body,
        out_shape=jax.ShapeDtypeStruct((N, D), jnp.float32),
    )(x, xi, acc, wri, bri, wout, bout, u, wrf, brf)


def _dense_ba_body(x_ref, xi_ref, acc_ref, wri_ref, bri_ref, wout_ref, bout_ref,
                   u_ref, wrf_ref, brf_ref, wi2_ref, bi2_ref, wj2_ref, bj2_ref,
                   out_ref, xi2_ref, xjt2_ref):
    x = _finish_block(x_ref[...], xi_ref[...], acc_ref, wri_ref, bri_ref,
                      wout_ref, bout_ref, u_ref, wrf_ref, brf_ref)
    out_ref[...] = x
    xa = _ssp(x)
    xi2_ref[...] = jnp.dot(xa, wi2_ref[...], preferred_element_type=jnp.float32) + bi2_ref[...]
    xjt2_ref[...] = jnp.dot(xa, wj2_ref[...], preferred_element_type=jnp.float32) + bj2_ref[...]


def _dense_ba_call(x, xi, acc, wri, bri, wout, bout, u, wrf, brf, wi2, bi2, wj2, bj2):
    return pl.pallas_call(
        _dense_ba_body,
        out_shape=(
            jax.ShapeDtypeStruct((N, D), jnp.float32),
            jax.ShapeDtypeStruct((N, D), jnp.float32),
            jax.ShapeDtypeStruct((N, D), jnp.float32),
        ),
    )(x, xi, acc, wri, bri, wout, bout, u, wrf, brf, wi2, bi2, wj2, bj2)


def _make_idx_cat(idx_i, idx_j):
    idx_cat = jnp.stack([idx_j.reshape(NW, N_BATCH, Q),
                         idx_i.reshape(NW, N_BATCH, Q)], axis=2)
    pad = jnp.zeros((NW, NGRP * GRP - N_BATCH, 2, Q), jnp.int32)
    return jnp.concatenate([idx_cat, pad], axis=1).reshape(NW, NGRP, GRP, 2, Q)


def kernel(features, distances, cutoffs, rbfs, idx_i, idx_j, W_rbf, W_i, b_i,
           W_j, b_j, Wri, bri, W_out, b_out, u, Wrf, brf):
    del distances
    x = features
    idx_cat = _make_idx_cat(idx_i, idx_j)
    g_all = _g_call(cutoffs, rbfs, W_rbf)
    xi, xjt = _dense_a_call(x, W_i[0], b_i[0].reshape(1, D),
                            W_j[0], b_j[0].reshape(1, D))
    outs = []
    for blk in range(B):
        acc = _sc_call(g_all[blk], xjt, idx_cat)
        bargs = (Wri[blk].reshape(2 * RI, D, D),
                 bri[blk].reshape(2 * RI, 1, D),
                 W_out[blk], b_out[blk].reshape(1, D),
                 u[blk].reshape(1, D),
                 Wrf[blk].reshape(2 * RF, D, D),
                 brf[blk].reshape(2 * RF, 1, D))
        if blk + 1 < B:
            x, xi, xjt = _dense_ba_call(x, xi, acc, *bargs,
                                        W_i[blk + 1], b_i[blk + 1].reshape(1, D),
                                        W_j[blk + 1], b_j[blk + 1].reshape(1, D))
        else:
            x = _dense_b_call(x, xi, acc, *bargs)
        outs.append(x)
    return jnp.stack(outs)


# speedup vs baseline: 1.1901x; 1.1901x over previous
import jax
import jax.numpy as jnp
from jax import lax
from jax.experimental import pallas as pl
from jax.experimental.pallas import tpu as pltpu
from jax.experimental.pallas import tpu_sc as plsc

N = 10000
P = 320000
D = 128
K = 64
B = 5
RI = 3
RF = 2

NC = 2
NS = 16
NW = NC * NS
Q = 80
PAIRS_PER_WORKER = P // NW
N_BATCH = PAIRS_PER_WORKER // Q
GRP = 6
NGRP = 21
RING = 3
N_MACRO = 20
GT = 4000
STRIPE = 624
ZCH = 24
TAIL = N - NS * STRIPE

SC_G_SHAPE = (P // 2, D)
SC_G_DTYPE = jnp.int32
SC_XJT_SHAPE = (N, D)


def _ssp(x):
    return jnp.maximum(x, 0.0) + jnp.log1p(jnp.exp(-jnp.abs(x))) - 0.6931471805599453


def _pack_bf16(v):
    lo = jax.lax.bitcast_convert_type(v[:, :D // 2].astype(jnp.bfloat16), jnp.uint16).astype(jnp.int32)
    hi = jax.lax.bitcast_convert_type(v[:, D // 2:].astype(jnp.bfloat16), jnp.uint16).astype(jnp.int32)
    w = lo | (hi << 16)
    w3 = w.reshape(w.shape[0] // 2, 2, D // 2)
    return jnp.concatenate([w3[:, 0, :], w3[:, 1, :]], axis=-1)


def _g_body(cut_ref, rbf_ref, wr_ref, g_ref):
    desc = cut_ref[...] * rbf_ref[...]
    g_ref[...] = _pack_bf16(jnp.dot(desc, wr_ref[...], preferred_element_type=jnp.float32))


def _g_call(cutoffs, rbfs, wr):
    T = GT
    grid = P // T
    return pl.pallas_call(
        _g_body,
        grid=(grid,),
        in_specs=[
            pl.BlockSpec((T, 1), lambda i: (i, 0)),
            pl.BlockSpec((T, K), lambda i: (i, 0)),
            pl.BlockSpec((K, D), lambda i: (0, 0)),
        ],
        out_specs=pl.BlockSpec((T // 2, D), lambda i: (i, 0)),
        out_shape=jax.ShapeDtypeStruct((P // 2, D), jnp.int32),
    )(cutoffs, rbfs, wr)


def _dense_a_body(x_ref, wi_ref, bi_ref, wj_ref, bj_ref, xi_ref, xjt_ref):
    xa = _ssp(x_ref[...])
    xi_ref[...] = jnp.dot(xa, wi_ref[...], preferred_element_type=jnp.float32) + bi_ref[...]
    xjt_ref[...] = jnp.dot(xa, wj_ref[...], preferred_element_type=jnp.float32) + bj_ref[...]


def _dense_a_call(x, wi, bi, wj, bj):
    return pl.pallas_call(
        _dense_a_body,
        out_shape=(
            jax.ShapeDtypeStruct((N, D), jnp.float32),
            jax.ShapeDtypeStruct((N, D), jnp.float32),
        ),
    )(x, wi, bi, wj, bj)


def _sc_body(g_hbm, xjt_hbm, idxc_hbm, out_hbm,
             idxj_s0, idxj_s1, idxj_s2,
             idxi_s0, idxi_s1, idxi_s2,
             idx_g, rows_v, g_v, zero_v, acc_sh,
             sem_idx, sem_r, sem_g, sem_s):
    idxj_s = (idxj_s0, idxj_s1, idxj_s2)
    idxi_s = (idxi_s0, idxi_s1, idxi_s2)
    c = lax.axis_index("c")
    s = lax.axis_index("s")
    wid = c * NS + s
    base = wid * PAIRS_PER_WORKER
    gbase = wid * (PAIRS_PER_WORKER // 2)

    def _zf(i, _):
        for j in range(D // 16):
            zero_v[i, pl.ds(j * 16, 16)] = jnp.zeros((16,), jnp.float32)
        return 0
    lax.fori_loop(0, ZCH, _zf, 0)
    for k in range(STRIPE // ZCH):
        pltpu.async_copy(zero_v, acc_sh.at[pl.ds(s * STRIPE + k * ZCH, ZCH)], sem_s)

    @pl.when(s == NS - 1)
    def _zero_tail():
        pltpu.async_copy(zero_v.at[pl.ds(0, TAIL)],
                         acc_sh.at[pl.ds(NS * STRIPE, TAIL)], sem_s)

    for k in range(STRIPE // ZCH):
        pltpu.make_async_copy(zero_v, acc_sh.at[pl.ds(0, ZCH)], sem_s).wait()

    @pl.when(s == NS - 1)
    def _zero_tail_wait():
        pltpu.make_async_copy(zero_v.at[pl.ds(0, TAIL)],
                              acc_sh.at[pl.ds(0, TAIL)], sem_s).wait()

    plsc.subcore_barrier()

    def _start_idx(grp):
        pltpu.async_copy(idxc_hbm.at[wid, grp], idx_g.at[grp % 2], sem_idx)

    def _wait_idx():
        pltpu.make_async_copy(idxc_hbm.at[0, 0], idx_g.at[0], sem_idx).wait()

    def _stage(bn, ring):
        slot = (bn // GRP) % 2
        bb = bn % GRP
        for t in range(Q // 16):
            sl = pl.ds(t * 16, 16)
            idxj_s[ring][sl] = idx_g[slot, bb, 0, sl]
            idxi_s[ring][sl] = idx_g[slot, bb, 1, sl]

    def _start_loads(bn, ring, gslot):
        pltpu.async_copy(xjt_hbm.at[idxj_s[ring]], rows_v.at[ring], sem_r)
        pltpu.async_copy(g_hbm.at[pl.ds(gbase + bn * (Q // 2), Q // 2)],
                         g_v.at[gslot], sem_g)

    def _wait_loads(ring, gslot):
        pltpu.make_async_copy(xjt_hbm.at[idxj_s[0]], rows_v.at[ring], sem_r).wait()
        pltpu.make_async_copy(g_hbm.at[pl.ds(0, Q // 2)], g_v.at[gslot], sem_g).wait()

    def _start_scatter(ring):
        pltpu.async_copy(rows_v.at[ring], acc_sh.at[idxi_s[ring]], sem_s, add=True)

    def _wait_scatter(ring):
        pltpu.make_async_copy(rows_v.at[ring], acc_sh.at[idxi_s[0]], sem_s).wait()

    def _mul(ring, gslot):
        @plsc.parallel_loop(0, Q // 2, unroll=4)
        def _rowpair(m):
            for sub in range(2):
                r = m * 2 + sub
                for j in range(D // 32):
                    wg = g_v[gslot, m, pl.ds(64 * sub + 16 * j, 16)]
                    glo = plsc.bitcast(wg << 16, jnp.float32)
                    ghi = plsc.bitcast(wg & -65536, jnp.float32)
                    lo = pl.ds(16 * j, 16)
                    hi = pl.ds(64 + 16 * j, 16)
                    rows_v[ring, r, lo] = rows_v[ring, r, lo] * glo
                    rows_v[ring, r, hi] = rows_v[ring, r, hi] * ghi

    _start_idx(0)
    _wait_idx()
    _stage(0, 0)
    _start_loads(0, 0, 0)

    def _macro(m, _):
        for b in range(GRP):
            bi = m * GRP + b
            ring = b % RING
            gslot = b % 2
            ring_n = (b + 1) % RING
            gslot_n = (b + 1) % 2
            if b == 0:
                _start_idx(m + 1)
            if b < 2:
                @pl.when(m > 0)
                def _ws():
                    _wait_scatter(ring_n)
            else:
                if b == GRP - 1:
                    _wait_idx()
                _wait_scatter(ring_n)
            _stage(bi + 1, ring_n)
            _start_loads(bi + 1, ring_n, gslot_n)
            _wait_loads(ring, gslot)
            _mul(ring, gslot)
            _start_scatter(ring)
        return 0

    lax.fori_loop(0, N_MACRO, _macro, 0)

    for bi in range(N_MACRO * GRP, N_BATCH):
        ring = bi % RING
        gslot = bi % 2
        if bi + 1 < N_BATCH:
            _wait_scatter((bi + 1) % RING)
            _stage(bi + 1, (bi + 1) % RING)
            _start_loads(bi + 1, (bi + 1) % RING, (bi + 1) % 2)
        _wait_loads(ring, gslot)
        _mul(ring, gslot)
        _start_scatter(ring)
    for ring in ((N_BATCH - 3) % RING, (N_BATCH - 2) % RING, (N_BATCH - 1) % RING):
        _wait_scatter(ring)

    plsc.subcore_barrier()
    pltpu.sync_copy(acc_sh.at[pl.ds(s * STRIPE, STRIPE)],
                    out_hbm.at[c, pl.ds(s * STRIPE, STRIPE)])

    @pl.when(s == NS - 1)
    def _out_tail():
        pltpu.sync_copy(acc_sh.at[pl.ds(NS * STRIPE, TAIL)],
                        out_hbm.at[c, pl.ds(NS * STRIPE, TAIL)])


def _sc_call(g, xjt, idx_cat):
    mesh = plsc.VectorSubcoreMesh(core_axis_name="c", subcore_axis_name="s",
                                  num_cores=NC, num_subcores=NS)
    return pl.kernel(
        _sc_body,
        out_type=jax.ShapeDtypeStruct((NC, N, D), jnp.float32),
        mesh=mesh,
        compiler_params=pltpu.CompilerParams(needs_layout_passes=False),
        scratch_types=[
            pltpu.VMEM((Q,), jnp.int32),
            pltpu.VMEM((Q,), jnp.int32),
            pltpu.VMEM((Q,), jnp.int32),
            pltpu.VMEM((Q,), jnp.int32),
            pltpu.VMEM((Q,), jnp.int32),
            pltpu.VMEM((Q,), jnp.int32),
            pltpu.VMEM((2, GRP, 2, Q), jnp.int32),
            pltpu.VMEM((RING, Q, D), jnp.float32),
            pltpu.VMEM((2, Q // 2, D), jnp.int32),
            pltpu.VMEM((ZCH, D), jnp.float32),
            pltpu.VMEM_SHARED((N, D), jnp.float32),
            pltpu.SemaphoreType.DMA,
            pltpu.SemaphoreType.DMA,
            pltpu.SemaphoreType.DMA,
            pltpu.SemaphoreType.DMA,
        ],
    )(g, xjt, idx_cat)


def _dense_b_body(x_ref, xi_ref, acc_ref, wri_ref, bri_ref, wout_ref, bout_ref,
                  u_ref, wrf_ref, brf_ref, out_ref):
    m = xi_ref[...] + acc_ref[0] + acc_ref[1]
    for r in range(RI):
        y = jnp.dot(_ssp(m), wri_ref[2 * r], preferred_element_type=jnp.float32) + bri_ref[2 * r]
        y = jnp.dot(_ssp(y), wri_ref[2 * r + 1], preferred_element_type=jnp.float32) + bri_ref[2 * r + 1]
        m = m + y
    m = _ssp(m)
    x = u_ref[...] * x_ref[...] + jnp.dot(m, wout_ref[...], preferred_element_type=jnp.float32) + bout_ref[...]
    for r in range(RF):
        y = jnp.dot(_ssp(x), wrf_ref[2 * r], preferred_element_type=jnp.float32) + brf_ref[2 * r]
        y = jnp.dot(_ssp(y), wrf_ref[2 * r + 1], preferred_element_type=jnp.float32) + brf_ref[2 * r + 1]
        x = x + y
    out_ref[...] = x


def _dense_b_call(x, xi, acc, wri, bri, wout, bout, u, wrf, brf):
    return pl.pallas_call(
        _dense_b_body,
        out_shape=jax.ShapeDtypeStruct((N, D), jnp.float32),
    )(x, xi, acc, wri, bri, wout, bout, u, wrf, brf)


def _make_idx_cat(idx_i, idx_j):
    idx_cat = jnp.stack([idx_j.reshape(NW, N_BATCH, Q),
                         idx_i.reshape(NW, N_BATCH, Q)], axis=2)
    pad = jnp.zeros((NW, NGRP * GRP - N_BATCH, 2, Q), jnp.int32)
    return jnp.concatenate([idx_cat, pad], axis=1).reshape(NW, NGRP, GRP, 2, Q)


def kernel(features, distances, cutoffs, rbfs, idx_i, idx_j, W_rbf, W_i, b_i,
           W_j, b_j, Wri, bri, W_out, b_out, u, Wrf, brf):
    del distances
    x = features
    idx_cat = _make_idx_cat(idx_i, idx_j)
    outs = []
    for blk in range(B):
        g = _g_call(cutoffs, rbfs, W_rbf[blk])
        xi, xjt = _dense_a_call(x, W_i[blk], b_i[blk].reshape(1, D),
                                W_j[blk], b_j[blk].reshape(1, D))
        acc = _sc_call(g, xjt, idx_cat)
        x = _dense_b_call(x, xi, acc,
                          Wri[blk].reshape(2 * RI, D, D),
                          bri[blk].reshape(2 * RI, 1, D),
                          W_out[blk], b_out[blk].reshape(1, D),
                          u[blk].reshape(1, D),
                          Wrf[blk].reshape(2 * RF, D, D),
                          brf[blk].reshape(2 * RF, 1, D))
        outs.append(x)
    return jnp.stack(outs)


# speedup vs baseline: 1.1906x; 1.0005x over previous
import jax
import jax.numpy as jnp
from jax import lax
from jax.experimental import pallas as pl
from jax.experimental.pallas import tpu as pltpu
from jax.experimental.pallas import tpu_sc as plsc

N = 10000
P = 320000
D = 128
K = 64
B = 5
RI = 3
RF = 2

NC = 2
NS = 16
NW = NC * NS
Q = 80
PAIRS_PER_WORKER = P // NW
N_BATCH = PAIRS_PER_WORKER // Q
GRP = 6
NGRP = 21
RING = 3
N_MACRO = 20
GT = 4000
STRIPE = 624
ZCH = 24
TAIL = N - NS * STRIPE

SC_G_SHAPE = (P // 2, D)
SC_G_DTYPE = jnp.int32
SC_XJT_SHAPE = (N, D)


def _ssp(x):
    return jnp.maximum(x, 0.0) + jnp.log1p(jnp.exp(-jnp.abs(x))) - 0.6931471805599453


def _pack_bf16(v):
    lo = jax.lax.bitcast_convert_type(v[:, :D // 2].astype(jnp.bfloat16), jnp.uint16).astype(jnp.int32)
    hi = jax.lax.bitcast_convert_type(v[:, D // 2:].astype(jnp.bfloat16), jnp.uint16).astype(jnp.int32)
    w = lo | (hi << 16)
    w3 = w.reshape(w.shape[0] // 2, 2, D // 2)
    return jnp.concatenate([w3[:, 0, :], w3[:, 1, :]], axis=-1)


def _g_body(cut_ref, rbf_ref, wr_ref, g_ref):
    desc = cut_ref[...] * rbf_ref[...]
    g_ref[...] = _pack_bf16(jnp.dot(desc, wr_ref[...], preferred_element_type=jnp.float32))


def _g_call(cutoffs, rbfs, wr):
    T = GT
    grid = P // T
    return pl.pallas_call(
        _g_body,
        grid=(grid,),
        in_specs=[
            pl.BlockSpec((T, 1), lambda i: (i, 0)),
            pl.BlockSpec((T, K), lambda i: (i, 0)),
            pl.BlockSpec((K, D), lambda i: (0, 0)),
        ],
        out_specs=pl.BlockSpec((T // 2, D), lambda i: (i, 0)),
        out_shape=jax.ShapeDtypeStruct((P // 2, D), jnp.int32),
    )(cutoffs, rbfs, wr)


def _dense_a_body(x_ref, wi_ref, bi_ref, wj_ref, bj_ref, xi_ref, xjt_ref):
    xa = _ssp(x_ref[...])
    xi_ref[...] = jnp.dot(xa, wi_ref[...], preferred_element_type=jnp.float32) + bi_ref[...]
    xjt_ref[...] = jnp.dot(xa, wj_ref[...], preferred_element_type=jnp.float32) + bj_ref[...]


def _dense_a_call(x, wi, bi, wj, bj):
    return pl.pallas_call(
        _dense_a_body,
        out_shape=(
            jax.ShapeDtypeStruct((N, D), jnp.float32),
            jax.ShapeDtypeStruct((N, D), jnp.float32),
        ),
    )(x, wi, bi, wj, bj)


def _sc_body(g_hbm, xjt_hbm, idxc_hbm, out_hbm,
             idxj_s0, idxj_s1, idxj_s2,
             idxi_s0, idxi_s1, idxi_s2,
             idx_g, rows_v, g_v, zero_v, acc_sh,
             sem_idx, sem_r, sem_g, sem_s):
    idxj_s = (idxj_s0, idxj_s1, idxj_s2)
    idxi_s = (idxi_s0, idxi_s1, idxi_s2)
    c = lax.axis_index("c")
    s = lax.axis_index("s")
    wid = c * NS + s
    base = wid * PAIRS_PER_WORKER
    gbase = wid * (PAIRS_PER_WORKER // 2)

    def _zf(i, _):
        for j in range(D // 16):
            zero_v[i, pl.ds(j * 16, 16)] = jnp.zeros((16,), jnp.float32)
        return 0
    lax.fori_loop(0, ZCH, _zf, 0)
    for k in range(STRIPE // ZCH):
        pltpu.async_copy(zero_v, acc_sh.at[pl.ds(s * STRIPE + k * ZCH, ZCH)], sem_s)

    @pl.when(s == NS - 1)
    def _zero_tail():
        pltpu.async_copy(zero_v.at[pl.ds(0, TAIL)],
                         acc_sh.at[pl.ds(NS * STRIPE, TAIL)], sem_s)

    for k in range(STRIPE // ZCH):
        pltpu.make_async_copy(zero_v, acc_sh.at[pl.ds(0, ZCH)], sem_s).wait()

    @pl.when(s == NS - 1)
    def _zero_tail_wait():
        pltpu.make_async_copy(zero_v.at[pl.ds(0, TAIL)],
                              acc_sh.at[pl.ds(0, TAIL)], sem_s).wait()

    plsc.subcore_barrier()

    def _start_idx(grp):
        pltpu.async_copy(idxc_hbm.at[wid, grp], idx_g.at[grp % 2], sem_idx)

    def _wait_idx():
        pltpu.make_async_copy(idxc_hbm.at[0, 0], idx_g.at[0], sem_idx).wait()

    def _stage(bn, ring):
        slot = (bn // GRP) % 2
        bb = bn % GRP
        for t in range(Q // 16):
            sl = pl.ds(t * 16, 16)
            idxj_s[ring][sl] = idx_g[slot, bb, 0, sl]
            idxi_s[ring][sl] = idx_g[slot, bb, 1, sl]

    def _start_loads(bn, ring, gslot):
        pltpu.async_copy(xjt_hbm.at[idxj_s[ring]], rows_v.at[ring], sem_r)
        pltpu.async_copy(g_hbm.at[pl.ds(gbase + bn * (Q // 2), Q // 2)],
                         g_v.at[gslot], sem_g)

    def _wait_loads(ring, gslot):
        pltpu.make_async_copy(xjt_hbm.at[idxj_s[0]], rows_v.at[ring], sem_r).wait()
        pltpu.make_async_copy(g_hbm.at[pl.ds(0, Q // 2)], g_v.at[gslot], sem_g).wait()

    def _start_scatter(ring):
        pltpu.async_copy(rows_v.at[ring], acc_sh.at[idxi_s[ring]], sem_s, add=True)

    def _wait_scatter(ring):
        pltpu.make_async_copy(rows_v.at[ring], acc_sh.at[idxi_s[0]], sem_s).wait()

    def _mul(ring, gslot):
        @plsc.parallel_loop(0, Q // 2, unroll=8)
        def _rowpair(m):
            for sub in range(2):
                r = m * 2 + sub
                for j in range(D // 32):
                    wg = g_v[gslot, m, pl.ds(64 * sub + 16 * j, 16)]
                    glo = plsc.bitcast(wg << 16, jnp.float32)
                    ghi = plsc.bitcast(wg & -65536, jnp.float32)
                    lo = pl.ds(16 * j, 16)
                    hi = pl.ds(64 + 16 * j, 16)
                    rows_v[ring, r, lo] = rows_v[ring, r, lo] * glo
                    rows_v[ring, r, hi] = rows_v[ring, r, hi] * ghi

    _start_idx(0)
    _wait_idx()
    _stage(0, 0)
    _start_loads(0, 0, 0)

    def _macro(m, _):
        for b in range(GRP):
            bi = m * GRP + b
            ring = b % RING
            gslot = b % 2
            ring_n = (b + 1) % RING
            gslot_n = (b + 1) % 2
            if b == 0:
                _start_idx(m + 1)
            if b < 2:
                @pl.when(m > 0)
                def _ws():
                    _wait_scatter(ring_n)
            else:
                if b == GRP - 1:
                    _wait_idx()
                _wait_scatter(ring_n)
            _stage(bi + 1, ring_n)
            _start_loads(bi + 1, ring_n, gslot_n)
            _wait_loads(ring, gslot)
            _mul(ring, gslot)
            _start_scatter(ring)
        return 0

    lax.fori_loop(0, N_MACRO, _macro, 0)

    for bi in range(N_MACRO * GRP, N_BATCH):
        ring = bi % RING
        gslot = bi % 2
        if bi + 1 < N_BATCH:
            _wait_scatter((bi + 1) % RING)
            _stage(bi + 1, (bi + 1) % RING)
            _start_loads(bi + 1, (bi + 1) % RING, (bi + 1) % 2)
        _wait_loads(ring, gslot)
        _mul(ring, gslot)
        _start_scatter(ring)
    for ring in ((N_BATCH - 3) % RING, (N_BATCH - 2) % RING, (N_BATCH - 1) % RING):
        _wait_scatter(ring)

    plsc.subcore_barrier()
    pltpu.sync_copy(acc_sh.at[pl.ds(s * STRIPE, STRIPE)],
                    out_hbm.at[c, pl.ds(s * STRIPE, STRIPE)])

    @pl.when(s == NS - 1)
    def _out_tail():
        pltpu.sync_copy(acc_sh.at[pl.ds(NS * STRIPE, TAIL)],
                        out_hbm.at[c, pl.ds(NS * STRIPE, TAIL)])


def _sc_call(g, xjt, idx_cat):
    mesh = plsc.VectorSubcoreMesh(core_axis_name="c", subcore_axis_name="s",
                                  num_cores=NC, num_subcores=NS)
    return pl.kernel(
        _sc_body,
        out_type=jax.ShapeDtypeStruct((NC, N, D), jnp.float32),
        mesh=mesh,
        compiler_params=pltpu.CompilerParams(needs_layout_passes=False),
        scratch_types=[
            pltpu.VMEM((Q,), jnp.int32),
            pltpu.VMEM((Q,), jnp.int32),
            pltpu.VMEM((Q,), jnp.int32),
            pltpu.VMEM((Q,), jnp.int32),
            pltpu.VMEM((Q,), jnp.int32),
            pltpu.VMEM((Q,), jnp.int32),
            pltpu.VMEM((2, GRP, 2, Q), jnp.int32),
            pltpu.VMEM((RING, Q, D), jnp.float32),
            pltpu.VMEM((2, Q // 2, D), jnp.int32),
            pltpu.VMEM((ZCH, D), jnp.float32),
            pltpu.VMEM_SHARED((N, D), jnp.float32),
            pltpu.SemaphoreType.DMA,
            pltpu.SemaphoreType.DMA,
            pltpu.SemaphoreType.DMA,
            pltpu.SemaphoreType.DMA,
        ],
    )(g, xjt, idx_cat)


def _dense_b_body(x_ref, xi_ref, acc_ref, wri_ref, bri_ref, wout_ref, bout_ref,
                  u_ref, wrf_ref, brf_ref, out_ref):
    m = xi_ref[...] + acc_ref[0] + acc_ref[1]
    for r in range(RI):
        y = jnp.dot(_ssp(m), wri_ref[2 * r], preferred_element_type=jnp.float32) + bri_ref[2 * r]
        y = jnp.dot(_ssp(y), wri_ref[2 * r + 1], preferred_element_type=jnp.float32) + bri_ref[2 * r + 1]
        m = m + y
    m = _ssp(m)
    x = u_ref[...] * x_ref[...] + jnp.dot(m, wout_ref[...], preferred_element_type=jnp.float32) + bout_ref[...]
    for r in range(RF):
        y = jnp.dot(_ssp(x), wrf_ref[2 * r], preferred_element_type=jnp.float32) + brf_ref[2 * r]
        y = jnp.dot(_ssp(y), wrf_ref[2 * r + 1], preferred_element_type=jnp.float32) + brf_ref[2 * r + 1]
        x = x + y
    out_ref[...] = x


def _dense_b_call(x, xi, acc, wri, bri, wout, bout, u, wrf, brf):
    return pl.pallas_call(
        _dense_b_body,
        out_shape=jax.ShapeDtypeStruct((N, D), jnp.float32),
    )(x, xi, acc, wri, bri, wout, bout, u, wrf, brf)


def _make_idx_cat(idx_i, idx_j):
    idx_cat = jnp.stack([idx_j.reshape(NW, N_BATCH, Q),
                         idx_i.reshape(NW, N_BATCH, Q)], axis=2)
    pad = jnp.zeros((NW, NGRP * GRP - N_BATCH, 2, Q), jnp.int32)
    return jnp.concatenate([idx_cat, pad], axis=1).reshape(NW, NGRP, GRP, 2, Q)


def kernel(features, distances, cutoffs, rbfs, idx_i, idx_j, W_rbf, W_i, b_i,
           W_j, b_j, Wri, bri, W_out, b_out, u, Wrf, brf):
    del distances
    x = features
    idx_cat = _make_idx_cat(idx_i, idx_j)
    outs = []
    for blk in range(B):
        g = _g_call(cutoffs, rbfs, W_rbf[blk])
        xi, xjt = _dense_a_call(x, W_i[blk], b_i[blk].reshape(1, D),
                                W_j[blk], b_j[blk].reshape(1, D))
        acc = _sc_call(g, xjt, idx_cat)
        x = _dense_b_call(x, xi, acc,
                          Wri[blk].reshape(2 * RI, D, D),
                          bri[blk].reshape(2 * RI, 1, D),
                          W_out[blk], b_out[blk].reshape(1, D),
                          u[blk].reshape(1, D),
                          Wrf[blk].reshape(2 * RF, D, D),
                          brf[blk].reshape(2 * RF, 1, D))
        outs.append(x)
    return jnp.stack(outs)
